# baseline (speedup 1.0000x reference)
import functools

import jax
import jax.numpy as jnp
from jax import lax
from jax.experimental import pallas as pl
from jax.experimental.pallas import tpu as pltpu
from jax.experimental.pallas import tpu_sc as plsc

NUM_EMBEDDINGS = 100000
EMBED_DIM = 64
B_ROWS = 4096
B_COLS = 100
TOTAL = B_ROWS * B_COLS

_info = plsc.get_sparse_core_info()
NC, NS = _info.num_cores, _info.num_subcores
NW = NC * NS

CHUNK = 128
NBUF = 5
PER_W = TOTAL // NW
N_CHUNKS = PER_W // CHUNK
N_GROUPS = N_CHUNKS // NBUF

_mesh = plsc.VectorSubcoreMesh(core_axis_name="c", subcore_axis_name="s")


@functools.partial(
    pl.kernel,
    mesh=_mesh,
    out_type=jax.ShapeDtypeStruct((TOTAL, 2 * EMBED_DIM), jnp.float32),
    scratch_types=[
        pltpu.VMEM((PER_W,), jnp.int32),
        pltpu.VMEM((NBUF, CHUNK, 2 * EMBED_DIM), jnp.float32),
        pltpu.SemaphoreType.DMA((NBUF,)),
        pltpu.SemaphoreType.DMA((NBUF,)),
    ],
    compiler_params=pltpu.CompilerParams(use_tc_tiling_on_sc=True),
)
def _sc_gather(idx_hbm, table_hbm, out_hbm, idx_v, rows_v, sem_g, sem_o):
    wid = lax.axis_index("s") * NC + lax.axis_index("c")
    base = wid * PER_W

    pltpu.sync_copy(idx_hbm.at[pl.ds(base, PER_W)], idx_v)

    def body(g, carry):
        goff = g * NBUF * CHUNK
        for b in range(NBUF):

            @pl.when(g > 0)
            def _wait_out():
                pltpu.make_async_copy(
                    rows_v.at[b], out_hbm.at[pl.ds(base, CHUNK)], sem_o.at[b]
                ).wait()

            pltpu.make_async_copy(
                table_hbm.at[idx_v.at[pl.ds(goff + b * CHUNK, CHUNK)]],
                rows_v.at[b],
                sem_g.at[b],
            ).start()
        for b in range(NBUF):
            off = goff + b * CHUNK
            pltpu.make_async_copy(
                table_hbm.at[idx_v.at[pl.ds(off, CHUNK)]],
                rows_v.at[b],
                sem_g.at[b],
            ).wait()
            pltpu.make_async_copy(
                rows_v.at[b], out_hbm.at[pl.ds(base + off, CHUNK)], sem_o.at[b]
            ).start()
        return carry

    lax.fori_loop(0, N_GROUPS, body, 0)

    for b in range(NBUF):
        pltpu.make_async_copy(
            rows_v.at[b], out_hbm.at[pl.ds(base, CHUNK)], sem_o.at[b]
        ).wait()


_TC_ROWS = 8
_TC_FLAT = _TC_ROWS * B_COLS


def _tc_extract_body(in_ref, out_ref):
    out_ref[...] = in_ref[:, :EMBED_DIM].reshape(_TC_ROWS, B_COLS, EMBED_DIM)


_tc_extract = pl.pallas_call(
    _tc_extract_body,
    grid=(B_ROWS // _TC_ROWS,),
    in_specs=[pl.BlockSpec((_TC_FLAT, 2 * EMBED_DIM), lambda i: (i, 0))],
    out_specs=pl.BlockSpec((_TC_ROWS, B_COLS, EMBED_DIM), lambda i: (i, 0, 0)),
    out_shape=jax.ShapeDtypeStruct((B_ROWS, B_COLS, EMBED_DIM), jnp.float32),
)


def kernel(emb, table):
    idx = emb.reshape(-1)
    table_p = jnp.pad(table, ((0, 0), (0, EMBED_DIM)))
    rows = _sc_gather(idx, table_p)
    return _tc_extract(rows)


# speedup vs baseline: 2.2940x; 2.2940x over previous
import functools

import jax
import jax.numpy as jnp
from jax import lax
from jax.experimental import pallas as pl
from jax.experimental.pallas import tpu as pltpu
from jax.experimental.pallas import tpu_sc as plsc

NUM_EMBEDDINGS = 100000
EMBED_DIM = 64
B_ROWS = 4096
B_COLS = 100

_info = plsc.get_sparse_core_info()
NC, NS = _info.num_cores, _info.num_subcores
NW = NC * NS

NBUF = 8
ROWS_W = B_ROWS // NW
N_GROUPS = ROWS_W // NBUF

_mesh = plsc.VectorSubcoreMesh(core_axis_name="c", subcore_axis_name="s")


@functools.partial(
    pl.kernel,
    mesh=_mesh,
    out_type=jax.ShapeDtypeStruct((B_ROWS, B_COLS, 2 * EMBED_DIM), jnp.float32),
    scratch_types=[
        pltpu.VMEM((ROWS_W * 128,), jnp.int32),
        pltpu.VMEM((NBUF, B_COLS, 2 * EMBED_DIM), jnp.float32),
        pltpu.SemaphoreType.DMA((NBUF,)),
        pltpu.SemaphoreType.DMA((NBUF,)),
    ],
    compiler_params=pltpu.CompilerParams(use_tc_tiling_on_sc=True),
)
def _sc_gather(idx_hbm, table_hbm, out_hbm, idx_v, rows_v, sem_g, sem_o):
    wid = lax.axis_index("s") * NC + lax.axis_index("c")
    base = wid * ROWS_W

    pltpu.sync_copy(idx_hbm.at[pl.ds(base * 128, ROWS_W * 128)], idx_v)

    def body(g, carry):
        grow = g * NBUF
        for b in range(NBUF):

            @pl.when(g > 0)
            def _wait_out():
                pltpu.make_async_copy(
                    rows_v.at[b], out_hbm.at[base], sem_o.at[b]
                ).wait()

            pltpu.make_async_copy(
                table_hbm.at[idx_v.at[pl.ds((grow + b) * 128, B_COLS)]],
                rows_v.at[b],
                sem_g.at[b],
            ).start()
        for b in range(NBUF):
            pltpu.make_async_copy(
                table_hbm.at[idx_v.at[pl.ds((grow + b) * 128, B_COLS)]],
                rows_v.at[b],
                sem_g.at[b],
            ).wait()
            pltpu.make_async_copy(
                rows_v.at[b], out_hbm.at[base + grow + b], sem_o.at[b]
            ).start()
        return carry

    lax.fori_loop(0, N_GROUPS, body, 0)

    for b in range(NBUF):
        pltpu.make_async_copy(
            rows_v.at[b], out_hbm.at[base], sem_o.at[b]
        ).wait()


def kernel(emb, table):
    emb_p = jnp.pad(emb, ((0, 0), (0, 128 - B_COLS))).reshape(-1)
    table_p = jnp.pad(table, ((0, 0), (0, EMBED_DIM)))
    rows3 = _sc_gather(emb_p, table_p)
    return rows3[:, :, :EMBED_DIM]
